# Initial kernel scaffold; baseline (speedup 1.0000x reference)
import jax
import jax.numpy as jnp
from jax.experimental import pallas as pl


def kernel(x, edge_index, edge_attr, batch, question_embedding, W_q, b_q, W0, root0, bias0, W1, root1, bias1, W2, root2, bias2, W3, root3, bias3):
    raise NotImplementedError("write your pallas kernel here")


# speedup vs baseline: 3.8146x; 3.8146x over previous
import functools

import jax
import jax.numpy as jnp
from jax import lax
from jax.experimental import pallas as pl
from jax.experimental.pallas import tpu as pltpu
from jax.experimental.pallas import tpu_sc as plsc

N = 10000
E = 320000
R = 8
D = 128
NC = 2
NS = 16
NW = NC * NS
EPW = E // NW
EPS = E // NS
C = 80
KPS = 5008
KEYP = KPS * NS
RPS = 640
CZ = 80
BN = 400
NB = N // BN

def _sc_prep_body(src_h, dst_h, attr_h, flat_h, norm_h,
                  counts_sh, sbuf, dbuf, abuf, kbuf, fbuf, ones_v, cbuf, nbuf,
                  zkbuf):
    c = lax.axis_index("c")
    s = lax.axis_index("s")
    wid = c * NS + s

    for j in range(C // 16):
        ones_v[pl.ds(j * 16, 16)] = jnp.ones((16,), jnp.float32)

    def zfill(i, _):
        zkbuf[pl.ds(i * 16, 16)] = jnp.zeros((16,), jnp.float32)
        return 0

    lax.fori_loop(0, KPS // 16, zfill, 0)
    pltpu.sync_copy(zkbuf, counts_sh.at[pl.ds(s * KPS, KPS)])
    plsc.subcore_barrier()

    def count_body(i, _):
        base = s * EPS + i * C
        pltpu.sync_copy(dst_h.at[pl.ds(base, C)], dbuf)
        pltpu.sync_copy(attr_h.at[pl.ds(base, C)], abuf)
        for j in range(C // 16):
            ds = pl.ds(j * 16, 16)
            kbuf[ds] = dbuf[ds] * R + abuf[ds]
        pltpu.sync_copy(ones_v, counts_sh.at[kbuf], add=True)
        return 0

    lax.fori_loop(0, EPS // C, count_body, 0)
    plsc.subcore_barrier()

    def edge_body(i, _):
        base = wid * EPW + i * C
        pltpu.sync_copy(src_h.at[pl.ds(base, C)], sbuf)
        pltpu.sync_copy(dst_h.at[pl.ds(base, C)], dbuf)
        pltpu.sync_copy(attr_h.at[pl.ds(base, C)], abuf)
        for j in range(C // 16):
            ds = pl.ds(j * 16, 16)
            kbuf[ds] = dbuf[ds] * R + abuf[ds]
            fbuf[ds] = abuf[ds] * N + sbuf[ds]
        pltpu.sync_copy(counts_sh.at[kbuf], cbuf)
        for j in range(C // 16):
            ds = pl.ds(j * 16, 16)
            nbuf[ds] = 1.0 / jnp.maximum(cbuf[ds], 1.0)
        pltpu.sync_copy(fbuf, flat_h.at[pl.ds(base, C)])
        pltpu.sync_copy(nbuf, norm_h.at[pl.ds(base, C)])
        return 0

    lax.fori_loop(0, EPW // C, edge_body, 0)


@functools.cache
def _get_sc_prep():
    mesh = plsc.VectorSubcoreMesh(
        core_axis_name="c", subcore_axis_name="s",
        num_cores=NC, num_subcores=NS)
    return pl.kernel(
        _sc_prep_body,
        out_type=(
            jax.ShapeDtypeStruct((E,), jnp.int32),
            jax.ShapeDtypeStruct((E,), jnp.float32),
        ),
        mesh=mesh,
        scratch_types=[
            pltpu.VMEM_SHARED((KEYP,), jnp.float32),
            pltpu.VMEM((C,), jnp.int32),
            pltpu.VMEM((C,), jnp.int32),
            pltpu.VMEM((C,), jnp.int32),
            pltpu.VMEM((C,), jnp.int32),
            pltpu.VMEM((C,), jnp.int32),
            pltpu.VMEM((C,), jnp.float32),
            pltpu.VMEM((C,), jnp.float32),
            pltpu.VMEM((C,), jnp.float32),
            pltpu.VMEM((KPS,), jnp.float32),
        ],
    )


def _sc_prep(src, dst, attr):
    return _get_sc_prep()(src, dst, attr)


def _sc_agg_body(h_h, flat_h, norm_h, dst_h, p_h,
                 acc_sh, fbuf, dbuf, nbuf, rows, sem):
    c = lax.axis_index("c")
    s = lax.axis_index("s")
    wid = c * NS + s
    nz = jnp.where(s == NS - 1, (N - (NS - 1) * RPS) // CZ, RPS // CZ)

    def zfill(i, _):
        for k in range(D // 16):
            rows[i, pl.ds(k * 16, 16)] = jnp.zeros((16,), jnp.float32)
        return 0

    lax.fori_loop(0, CZ, zfill, 0)

    def zcopy(j, _):
        pltpu.sync_copy(rows, acc_sh.at[pl.ds(s * RPS + j * CZ, CZ)])
        return 0

    lax.fori_loop(0, nz, zcopy, 0)
    plsc.subcore_barrier()

    def body(i, _):
        base = wid * EPW + i * C
        pltpu.sync_copy(flat_h.at[pl.ds(base, C)], fbuf)
        pltpu.sync_copy(dst_h.at[pl.ds(base, C)], dbuf)
        pltpu.sync_copy(norm_h.at[pl.ds(base, C)], nbuf)
        pltpu.async_copy(h_h.at[fbuf], rows, sem).wait()

        def scale(g, _):
            nv = nbuf[pl.ds(g * 16, 16)]
            for l in range(16):
                sc = nv[l]
                e = g * 16 + l
                for k in range(D // 16):
                    ds = pl.ds(k * 16, 16)
                    rows[e, ds] = rows[e, ds] * sc
            return 0

        lax.fori_loop(0, C // 16, scale, 0)
        pltpu.sync_copy(rows, acc_sh.at[dbuf], add=True)
        return 0

    lax.fori_loop(0, EPW // C, body, 0)
    plsc.subcore_barrier()

    def wcopy(j, _):
        off = s * RPS + j * CZ
        pltpu.sync_copy(acc_sh.at[pl.ds(off, CZ)], rows)
        pltpu.sync_copy(rows, p_h.at[c, pl.ds(off, CZ)])
        return 0

    lax.fori_loop(0, nz, wcopy, 0)


@functools.cache
def _get_sc_agg():
    mesh = plsc.VectorSubcoreMesh(
        core_axis_name="c", subcore_axis_name="s",
        num_cores=NC, num_subcores=NS)
    return pl.kernel(
        _sc_agg_body,
        out_type=jax.ShapeDtypeStruct((NC, N, D), jnp.float32),
        mesh=mesh,
        scratch_types=[
            pltpu.VMEM_SHARED((N, D), jnp.float32),
            pltpu.VMEM((C,), jnp.int32),
            pltpu.VMEM((C,), jnp.int32),
            pltpu.VMEM((C,), jnp.float32),
            pltpu.VMEM((C, D), jnp.float32),
            pltpu.SemaphoreType.DMA,
        ],
    )


def _sc_agg(h, flat, norm, dst):
    return _get_sc_agg()(h, flat, norm, dst)


def _dense0_body(x_ref, w_ref, root_ref, bias_ref, h_ref, rt_ref):
    xb = x_ref[...]
    for r in range(R):
        h_ref[r] = jnp.dot(xb, w_ref[r], preferred_element_type=jnp.float32)
    rt_ref[...] = (jnp.dot(xb, root_ref[...], preferred_element_type=jnp.float32)
                   + bias_ref[...])


def _dense_mid_body(p0_ref, p1_ref, rt_ref, w_ref, root_ref, bias_ref,
                    h_ref, rto_ref):
    hb = jnp.maximum(p0_ref[...] + p1_ref[...] + rt_ref[...], 0.0)
    for r in range(R):
        h_ref[r] = jnp.dot(hb, w_ref[r], preferred_element_type=jnp.float32)
    rto_ref[...] = (jnp.dot(hb, root_ref[...], preferred_element_type=jnp.float32)
                    + bias_ref[...])


def _dense_q_body(p0_ref, p1_ref, rt_ref, qe_ref, wq_ref, bq_ref, batch_ref,
                  wa_ref, wb_ref, ra_ref, rb_ref, bias_ref, h_ref, rto_ref):
    hb = jnp.maximum(p0_ref[...] + p1_ref[...] + rt_ref[...], 0.0)
    qx = jnp.maximum(
        jnp.dot(qe_ref[...], wq_ref[...], preferred_element_type=jnp.float32)
        + bq_ref[...], 0.0)
    bb = batch_ref[0, 0, :]
    oh = (bb[:, None] == lax.broadcasted_iota(jnp.int32, (1, 16), 1)
          ).astype(jnp.float32)
    qb = jnp.dot(oh, qx, preferred_element_type=jnp.float32)
    for r in range(R):
        h_ref[r] = (jnp.dot(hb, wa_ref[r], preferred_element_type=jnp.float32)
                    + jnp.dot(qb, wb_ref[r], preferred_element_type=jnp.float32))
    rto_ref[...] = (jnp.dot(hb, ra_ref[...], preferred_element_type=jnp.float32)
                    + jnp.dot(qb, rb_ref[...], preferred_element_type=jnp.float32)
                    + bias_ref[...])


def _final_body(p0_ref, p1_ref, rt_ref, o_ref):
    o_ref[...] = p0_ref[...] + p1_ref[...] + rt_ref[...]


def _full(shape):
    nd = len(shape)
    return pl.BlockSpec(shape, lambda i: (0,) * nd)


def _rows(d):
    return pl.BlockSpec((BN, d), lambda i: (i, 0))


_H_OUT = (jax.ShapeDtypeStruct((R, N, D), jnp.float32),
          jax.ShapeDtypeStruct((N, D), jnp.float32))
_H_SPECS = (pl.BlockSpec((R, BN, D), lambda i: (0, i, 0)), _rows(D))


def _dense0(x, w, root, bias):
    return pl.pallas_call(
        _dense0_body,
        grid=(NB,),
        in_specs=[_rows(D), _full((R, D, D)), _full((D, D)), _full((1, D))],
        out_specs=list(_H_SPECS),
        out_shape=list(_H_OUT),
    )(x, w, root, bias)


def _dense_mid(p0, p1, rt, w, root, bias):
    return pl.pallas_call(
        _dense_mid_body,
        grid=(NB,),
        in_specs=[_rows(D), _rows(D), _rows(D),
                  _full((R, D, D)), _full((D, D)), _full((1, D))],
        out_specs=list(_H_SPECS),
        out_shape=list(_H_OUT),
    )(p0, p1, rt, w, root, bias)


def _dense_q(p0, p1, rt, qe, wq, bq, batch3, wa, wb, ra, rb, bias):
    rq = wq.shape[1]
    return pl.pallas_call(
        _dense_q_body,
        grid=(NB,),
        in_specs=[_rows(D), _rows(D), _rows(D),
                  _full(qe.shape), _full(wq.shape), _full((1, rq)),
                  pl.BlockSpec((1, 1, BN), lambda i: (i, 0, 0)),
                  _full((R, D, D)), _full((R, rq, D)),
                  _full((D, D)), _full((rq, D)), _full((1, D))],
        out_specs=list(_H_SPECS),
        out_shape=list(_H_OUT),
    )(p0, p1, rt, qe, wq, bq, batch3, wa, wb, ra, rb, bias)


def _final(p0, p1, rt):
    return pl.pallas_call(
        _final_body,
        grid=(NB,),
        in_specs=[_rows(D), _rows(D), _rows(D)],
        out_specs=_rows(D),
        out_shape=jax.ShapeDtypeStruct((N, D), jnp.float32),
    )(p0, p1, rt)


def kernel(x, edge_index, edge_attr, batch, question_embedding,
           W_q, b_q, W0, root0, bias0, W1, root1, bias1,
           W2, root2, bias2, W3, root3, bias3):
    src = edge_index[0]
    dst = edge_index[1]

    flat, norm = _sc_prep(src, dst, edge_attr)

    h0, rt0 = _dense0(x, W0, root0, bias0.reshape(1, D))
    p0 = _sc_agg(h0.reshape(R * N, D), flat, norm, dst)

    h1, rt1 = _dense_q(
        p0[0], p0[1], rt0, question_embedding, W_q, b_q.reshape(1, -1),
        batch.reshape(NB, 1, BN), W1[:, :D, :], W1[:, D:, :],
        root1[:D, :], root1[D:, :], bias1.reshape(1, D))
    p1 = _sc_agg(h1.reshape(R * N, D), flat, norm, dst)

    h2, rt2 = _dense_mid(p1[0], p1[1], rt1, W2, root2, bias2.reshape(1, D))
    p2 = _sc_agg(h2.reshape(R * N, D), flat, norm, dst)

    h3, rt3 = _dense_mid(p2[0], p2[1], rt2, W3, root3, bias3.reshape(1, D))
    p3 = _sc_agg(h3.reshape(R * N, D), flat, norm, dst)

    return _final(p3[0], p3[1], rt3)
